# baseline (speedup 1.0000x reference)
import functools

import jax
import jax.numpy as jnp
from jax import lax
from jax.experimental import pallas as pl
from jax.experimental.pallas import tpu as pltpu
from jax.experimental.pallas import tpu_sc as plsc

N = 1048576
M = 1200
NC = 2
NS = 16
NW = NC * NS
L = 16
CHUNK = N // NW
SUB = 8192
NSUB = CHUNK // SUB
VPS = SUB // L
U = 32
YO = M + L
LS = 2 * M + 2 * L

_mesh = plsc.VectorSubcoreMesh(
    core_axis_name="c", subcore_axis_name="s", num_cores=NC, num_subcores=NS)


@functools.partial(
    pl.kernel,
    out_type=jax.ShapeDtypeStruct((NW, LS), jnp.float32),
    mesh=_mesh,
    compiler_params=pltpu.CompilerParams(needs_layout_passes=False),
    scratch_types=[
        pltpu.VMEM((L * LS,), jnp.float32),
        pltpu.VMEM((LS,), jnp.float32),
        pltpu.VMEM((2, SUB), jnp.int32),
        pltpu.VMEM((2, SUB), jnp.float32),
        pltpu.VMEM((2, SUB), jnp.float32),
        pltpu.SemaphoreType.DMA,
        pltpu.SemaphoreType.DMA,
    ],
)
def _phase_a(x_hbm, y_hbm, m_hbm, part_hbm, bins, red, buf_m, buf_x, buf_y,
             sem0, sem1):
    wid = lax.axis_index("s") * NC + lax.axis_index("c")
    base = wid * CHUNK
    sems = (sem0, sem1)

    neg = jnp.full((L,), -jnp.inf, jnp.float32)
    lane = lax.broadcasted_iota(jnp.int32, (L,), 0)
    lane_base = lane * LS

    def copies(s):
        par = s % 2
        off = base + s * SUB
        sem = sems[par]
        return [
            pltpu.make_async_copy(m_hbm.at[pl.ds(off, SUB)], buf_m.at[par], sem),
            pltpu.make_async_copy(x_hbm.at[pl.ds(off, SUB)], buf_x.at[par], sem),
            pltpu.make_async_copy(y_hbm.at[pl.ds(off, SUB)], buf_y.at[par], sem),
        ]

    def scan_step(par, voff, carry):
        cur_m, cmx, cmy = carry
        m = buf_m[par, pl.ds(voff, L)]
        x = buf_x[par, pl.ds(voff, L)]
        y = buf_y[par, pl.ds(voff, L)]
        changed = m != cur_m
        ix = lane_base + cur_m
        plsc.store_scatter(bins, [ix], cmx, mask=changed)
        plsc.store_scatter(bins, [ix + YO], cmy, mask=changed)
        cmx = jnp.where(changed, x, jnp.maximum(cmx, x))
        cmy = jnp.where(changed, y, jnp.maximum(cmy, y))
        return m, cmx, cmy

    for c in copies(0):
        c.start()

    @plsc.parallel_loop(0, LS // L, 1, unroll=8)
    def _(g):
        red[pl.ds(g * L, L)] = neg

    carry = None
    first = None
    prev_last = None
    for s in range(NSUB):
        par = s % 2
        if s + 1 < NSUB:
            for c in copies(s + 1):
                c.start()
        for c in copies(s):
            c.wait()
        vlast = buf_m[par, pl.ds(SUB - L, L)]
        last_s = vlast[L - 1]
        if s == 0:
            vfirst = buf_m[0, pl.ds(0, L)]
            first = vfirst[0]
            start_col = (first >> 4) << 4
            carry = (vfirst, buf_x[0, pl.ds(0, L)], buf_y[0, pl.ds(0, L)])
            t0 = 1
        else:
            start_col = prev_last + 1
            t0 = 0

        nvec = (last_s - start_col + L) >> 4
        def init_body(j, _, start_col=start_col):
            c0 = start_col + j * L
            for ln in range(L):
                bins[pl.ds(ln * LS + c0, L)] = neg
                bins[pl.ds(ln * LS + YO + c0, L)] = neg
            return 0
        lax.fori_loop(0, nvec, init_body, 0)
        prev_last = last_s

        def block_body(t, carry, par=par):
            return scan_step(par, t * L, carry)
        carry = plsc.parallel_loop(t0, VPS, 1, unroll=U, carry=carry)(block_body)

    cur_m, cmx, cmy = carry
    ix = lane_base + cur_m
    plsc.store_scatter(bins, [ix], cmx)
    plsc.store_scatter(bins, [ix + YO], cmy)

    for ln in range(L):
        bins[pl.ds(ln * LS + prev_last + 1, L)] = neg
        bins[pl.ds(ln * LS + YO + prev_last + 1, L)] = neg

    gstart = (first >> 4) << 4
    nred = ((prev_last - gstart) >> 4) + 1
    def red_body(j, _):
        c0 = gstart + j * L
        accx = neg
        accy = neg
        for ln in range(L):
            accx = jnp.maximum(accx, bins[pl.ds(ln * LS + c0, L)])
            accy = jnp.maximum(accy, bins[pl.ds(ln * LS + YO + c0, L)])
        red[pl.ds(c0, L)] = accx
        red[pl.ds(YO + c0, L)] = accy
        return 0
    lax.fori_loop(0, nred, red_body, 0)

    pltpu.sync_copy(red, part_hbm.at[wid])


def _phase_b_body(p_ref, o_ref):
    p = p_ref[...]
    mx = jnp.max(p[:, :M], axis=0)
    my = jnp.max(p[:, YO:YO + M], axis=0)
    d = mx - my
    o_ref[0] = jnp.sum(d * d) * (1.0 / M)


_phase_b = pl.pallas_call(
    _phase_b_body,
    out_shape=jax.ShapeDtypeStruct((1,), jnp.float32),
    out_specs=pl.BlockSpec(memory_space=pltpu.SMEM),
)


def kernel(output, target, months):
    x = output.reshape(-1)
    y = target.reshape(-1)
    part = _phase_a(x, y, months)
    loss = _phase_b(part)
    return loss.reshape(())


# speedup vs baseline: 1.5117x; 1.5117x over previous
import functools

import jax
import jax.numpy as jnp
from jax import lax
from jax.experimental import pallas as pl
from jax.experimental.pallas import tpu as pltpu
from jax.experimental.pallas import tpu_sc as plsc

N = 1048576
M = 1200
NC = 2
NS = 16
NW = NC * NS
L = 16
CHUNK = N // NW
SUB = 8192
NSUB = CHUNK // SUB
VPS = SUB // L
U = 8
YO = M + L
LS = 2 * M + 2 * L

_mesh = plsc.VectorSubcoreMesh(
    core_axis_name="c", subcore_axis_name="s", num_cores=NC, num_subcores=NS)


@functools.partial(
    pl.kernel,
    out_type=jax.ShapeDtypeStruct((NW, LS), jnp.float32),
    mesh=_mesh,
    compiler_params=pltpu.CompilerParams(needs_layout_passes=False),
    scratch_types=[
        pltpu.VMEM((L * LS,), jnp.float32),
        pltpu.VMEM((LS,), jnp.float32),
        pltpu.VMEM((2, SUB), jnp.int32),
        pltpu.VMEM((2, SUB), jnp.float32),
        pltpu.VMEM((2, SUB), jnp.float32),
        pltpu.SemaphoreType.DMA,
        pltpu.SemaphoreType.DMA,
    ],
)
def _phase_a(x_hbm, y_hbm, m_hbm, part_hbm, bins, red, buf_m, buf_x, buf_y,
             sem0, sem1):
    wid = lax.axis_index("s") * NC + lax.axis_index("c")
    base = wid * CHUNK
    sems = (sem0, sem1)

    neg = jnp.full((L,), -jnp.inf, jnp.float32)
    lane = lax.broadcasted_iota(jnp.int32, (L,), 0)
    lane_base = lane * LS

    def copies(s):
        par = s % 2
        off = base + s * SUB
        sem = sems[par]
        return [
            pltpu.make_async_copy(m_hbm.at[pl.ds(off, SUB)], buf_m.at[par], sem),
            pltpu.make_async_copy(x_hbm.at[pl.ds(off, SUB)], buf_x.at[par], sem),
            pltpu.make_async_copy(y_hbm.at[pl.ds(off, SUB)], buf_y.at[par], sem),
        ]

    def scan_step(par, voff, carry):
        cur_m, cmx, cmy = carry
        m = buf_m[par, pl.ds(voff, L)]
        x = buf_x[par, pl.ds(voff, L)]
        y = buf_y[par, pl.ds(voff, L)]
        changed = m != cur_m
        ix = lane_base + cur_m
        plsc.store_scatter(bins, [ix], cmx, mask=changed)
        plsc.store_scatter(bins, [ix + YO], cmy, mask=changed)
        cmx = jnp.where(changed, x, jnp.maximum(cmx, x))
        cmy = jnp.where(changed, y, jnp.maximum(cmy, y))
        return m, cmx, cmy

    for c in copies(0):
        c.start()

    @plsc.parallel_loop(0, LS // L, 1, unroll=8)
    def _(g):
        red[pl.ds(g * L, L)] = neg

    carry = None
    first = None
    prev_last = None
    for s in range(NSUB):
        par = s % 2
        if s + 1 < NSUB:
            for c in copies(s + 1):
                c.start()
        for c in copies(s):
            c.wait()
        vlast = buf_m[par, pl.ds(SUB - L, L)]
        last_s = vlast[L - 1]
        if s == 0:
            vfirst = buf_m[0, pl.ds(0, L)]
            first = vfirst[0]
            start_col = (first >> 4) << 4
            carry = (vfirst, buf_x[0, pl.ds(0, L)], buf_y[0, pl.ds(0, L)])
            t0 = 1
        else:
            start_col = prev_last + 1
            t0 = 0

        nvec = (last_s - start_col + L) >> 4
        def init_body(j, _, start_col=start_col):
            c0 = start_col + j * L
            for ln in range(L):
                bins[pl.ds(ln * LS + c0, L)] = neg
                bins[pl.ds(ln * LS + YO + c0, L)] = neg
            return 0
        lax.fori_loop(0, nvec, init_body, 0)
        prev_last = last_s

        def block_body(t, carry, par=par):
            return scan_step(par, t * L, carry)
        carry = plsc.parallel_loop(t0, VPS, 1, unroll=U, carry=carry)(block_body)

    cur_m, cmx, cmy = carry
    ix = lane_base + cur_m
    plsc.store_scatter(bins, [ix], cmx)
    plsc.store_scatter(bins, [ix + YO], cmy)

    for ln in range(L):
        bins[pl.ds(ln * LS + prev_last + 1, L)] = neg
        bins[pl.ds(ln * LS + YO + prev_last + 1, L)] = neg

    gstart = (first >> 4) << 4
    nred = ((prev_last - gstart) >> 4) + 1
    def red_body(j, _):
        c0 = gstart + j * L
        accx = neg
        accy = neg
        for ln in range(L):
            accx = jnp.maximum(accx, bins[pl.ds(ln * LS + c0, L)])
            accy = jnp.maximum(accy, bins[pl.ds(ln * LS + YO + c0, L)])
        red[pl.ds(c0, L)] = accx
        red[pl.ds(YO + c0, L)] = accy
        return 0
    lax.fori_loop(0, nred, red_body, 0)

    pltpu.sync_copy(red, part_hbm.at[wid])


def _phase_b_body(p_ref, o_ref):
    p = p_ref[...]
    mx = jnp.max(p[:, :M], axis=0)
    my = jnp.max(p[:, YO:YO + M], axis=0)
    d = mx - my
    o_ref[0] = jnp.sum(d * d) * (1.0 / M)


_phase_b = pl.pallas_call(
    _phase_b_body,
    out_shape=jax.ShapeDtypeStruct((1,), jnp.float32),
    out_specs=pl.BlockSpec(memory_space=pltpu.SMEM),
)


def kernel(output, target, months):
    x = output.reshape(-1)
    y = target.reshape(-1)
    part = _phase_a(x, y, months)
    loss = _phase_b(part)
    return loss.reshape(())


# speedup vs baseline: 1.5134x; 1.0011x over previous
import functools

import jax
import jax.numpy as jnp
from jax import lax
from jax.experimental import pallas as pl
from jax.experimental.pallas import tpu as pltpu
from jax.experimental.pallas import tpu_sc as plsc

N = 1048576
M = 1200
NC = 2
NS = 16
NW = NC * NS
L = 16
CHUNK = N // NW
SUB = 8192
NSUB = CHUNK // SUB
VPS = SUB // L
U = 4
YO = M + L
LS = 2 * M + 2 * L

_mesh = plsc.VectorSubcoreMesh(
    core_axis_name="c", subcore_axis_name="s", num_cores=NC, num_subcores=NS)


@functools.partial(
    pl.kernel,
    out_type=jax.ShapeDtypeStruct((NW, LS), jnp.float32),
    mesh=_mesh,
    compiler_params=pltpu.CompilerParams(needs_layout_passes=False),
    scratch_types=[
        pltpu.VMEM((L * LS,), jnp.float32),
        pltpu.VMEM((LS,), jnp.float32),
        pltpu.VMEM((2, SUB), jnp.int32),
        pltpu.VMEM((2, SUB), jnp.float32),
        pltpu.VMEM((2, SUB), jnp.float32),
        pltpu.SemaphoreType.DMA,
        pltpu.SemaphoreType.DMA,
    ],
)
def _phase_a(x_hbm, y_hbm, m_hbm, part_hbm, bins, red, buf_m, buf_x, buf_y,
             sem0, sem1):
    wid = lax.axis_index("s") * NC + lax.axis_index("c")
    base = wid * CHUNK
    sems = (sem0, sem1)

    neg = jnp.full((L,), -jnp.inf, jnp.float32)
    lane = lax.broadcasted_iota(jnp.int32, (L,), 0)
    lane_base = lane * LS

    def copies(s):
        par = s % 2
        off = base + s * SUB
        sem = sems[par]
        return [
            pltpu.make_async_copy(m_hbm.at[pl.ds(off, SUB)], buf_m.at[par], sem),
            pltpu.make_async_copy(x_hbm.at[pl.ds(off, SUB)], buf_x.at[par], sem),
            pltpu.make_async_copy(y_hbm.at[pl.ds(off, SUB)], buf_y.at[par], sem),
        ]

    def scan_step(par, voff, carry):
        cur_m, cmx, cmy = carry
        m = buf_m[par, pl.ds(voff, L)]
        x = buf_x[par, pl.ds(voff, L)]
        y = buf_y[par, pl.ds(voff, L)]
        changed = m != cur_m
        ix = lane_base + cur_m
        plsc.store_scatter(bins, [ix], cmx, mask=changed)
        plsc.store_scatter(bins, [ix + YO], cmy, mask=changed)
        cmx = jnp.where(changed, x, jnp.maximum(cmx, x))
        cmy = jnp.where(changed, y, jnp.maximum(cmy, y))
        return m, cmx, cmy

    for c in copies(0):
        c.start()

    @plsc.parallel_loop(0, LS // L, 1, unroll=8)
    def _(g):
        red[pl.ds(g * L, L)] = neg

    carry = None
    first = None
    prev_last = None
    for s in range(NSUB):
        par = s % 2
        if s + 1 < NSUB:
            for c in copies(s + 1):
                c.start()
        for c in copies(s):
            c.wait()
        vlast = buf_m[par, pl.ds(SUB - L, L)]
        last_s = vlast[L - 1]
        if s == 0:
            vfirst = buf_m[0, pl.ds(0, L)]
            first = vfirst[0]
            start_col = (first >> 4) << 4
            carry = (vfirst, buf_x[0, pl.ds(0, L)], buf_y[0, pl.ds(0, L)])
            t0 = 1
        else:
            start_col = prev_last + 1
            t0 = 0

        nvec = (last_s - start_col + L) >> 4
        def init_body(j, _, start_col=start_col):
            c0 = start_col + j * L
            for ln in range(L):
                bins[pl.ds(ln * LS + c0, L)] = neg
                bins[pl.ds(ln * LS + YO + c0, L)] = neg
            return 0
        lax.fori_loop(0, nvec, init_body, 0)
        prev_last = last_s

        def block_body(t, carry, par=par):
            return scan_step(par, t * L, carry)
        carry = plsc.parallel_loop(t0, VPS, 1, unroll=U, carry=carry)(block_body)

    cur_m, cmx, cmy = carry
    ix = lane_base + cur_m
    plsc.store_scatter(bins, [ix], cmx)
    plsc.store_scatter(bins, [ix + YO], cmy)

    for ln in range(L):
        bins[pl.ds(ln * LS + prev_last + 1, L)] = neg
        bins[pl.ds(ln * LS + YO + prev_last + 1, L)] = neg

    gstart = (first >> 4) << 4
    nred = ((prev_last - gstart) >> 4) + 1
    def red_body(j, _):
        c0 = gstart + j * L
        accx = neg
        accy = neg
        for ln in range(L):
            accx = jnp.maximum(accx, bins[pl.ds(ln * LS + c0, L)])
            accy = jnp.maximum(accy, bins[pl.ds(ln * LS + YO + c0, L)])
        red[pl.ds(c0, L)] = accx
        red[pl.ds(YO + c0, L)] = accy
        return 0
    lax.fori_loop(0, nred, red_body, 0)

    pltpu.sync_copy(red, part_hbm.at[wid])


def _phase_b_body(p_ref, o_ref):
    p = p_ref[...]
    mx = jnp.max(p[:, :M], axis=0)
    my = jnp.max(p[:, YO:YO + M], axis=0)
    d = mx - my
    o_ref[0] = jnp.sum(d * d) * (1.0 / M)


_phase_b = pl.pallas_call(
    _phase_b_body,
    out_shape=jax.ShapeDtypeStruct((1,), jnp.float32),
    out_specs=pl.BlockSpec(memory_space=pltpu.SMEM),
)


def kernel(output, target, months):
    x = output.reshape(-1)
    y = target.reshape(-1)
    part = _phase_a(x, y, months)
    loss = _phase_b(part)
    return loss.reshape(())
